# Initial kernel scaffold; baseline (speedup 1.0000x reference)
import functools

import jax
import jax.numpy as jnp
from jax import lax
from jax.experimental import pallas as pl
from jax.experimental.pallas import tpu as pltpu
from jax.experimental.pallas import tpu_sc as plsc

N_NODES = 10000
FDIM = 128
NUM_EDGES = 320000
CHUNK = 128
NUM_CHUNKS = NUM_EDGES // CHUNK
NUM_CORES = 2
NUM_SUBCORES = 16
ROWS_PER_TILE = N_NODES // NUM_SUBCORES
ZROWS = 125

_DOT_DIMS = (((1,), (1,)), ((), ()))


def _dot(a, w):
    return lax.dot_general(a, w, _DOT_DIMS, preferred_element_type=jnp.float32)


def _stage_a_body(xm, xd, ewin, w1r, w1s, w2r, w2s, b1r, b2r,
                  ym1, ym2, root1, root2, ew):
    xmb = xm[...]
    xdb = xd[...]
    ym1[...] = _dot(xmb, w1r[...])
    ym2[...] = _dot(xmb, w2r[...])
    root1[...] = _dot(xmb, w1s[...]) + b1r[...]
    root2[...] = _dot(xdb, w2s[...]) + b2r[...]
    ew[...] = jax.nn.sigmoid(ewin[...])


def _stage_c_body(agg1, root1, agg2, root2, w3r, w3s, b3r, ym3, root3):
    movie = jnp.maximum(agg1[...] + root1[...], 0.0)
    ym3[...] = _dot(movie, w3r[...])
    user = jnp.maximum(agg2[...] + root2[...], 0.0)
    root3[...] = _dot(user, w3s[...]) + b3r[...]


def _stage_e_body(agg3, root3, wl, bl, out):
    a3 = agg3[...]
    user = jnp.maximum(a3[0] + a3[1] + root3[...], 0.0)
    out[...] = _dot(user, wl[...]) + bl[...]


_GRID = 10
_ROWB = N_NODES // _GRID

_node_spec = pl.BlockSpec((_ROWB, FDIM), lambda i: (i, 0))
_w_spec = pl.BlockSpec((FDIM, FDIM), lambda i: (0, 0))
_b_spec = pl.BlockSpec((1, FDIM), lambda i: (0, 0))
_ew_spec = pl.BlockSpec((NUM_EDGES // FDIM // _GRID, FDIM), lambda i: (i, 0))
_node_sds = jax.ShapeDtypeStruct((N_NODES, FDIM), jnp.float32)


def _stage_a(xm, xd, ew2d, w1r, w1s, w2r, w2s, b1r, b2r):
    return pl.pallas_call(
        _stage_a_body,
        grid=(_GRID,),
        in_specs=[_node_spec, _node_spec, _ew_spec,
                  _w_spec, _w_spec, _w_spec, _w_spec, _b_spec, _b_spec],
        out_specs=[_node_spec, _node_spec, _node_spec, _node_spec, _ew_spec],
        out_shape=[_node_sds, _node_sds, _node_sds, _node_sds,
                   jax.ShapeDtypeStruct((NUM_EDGES // FDIM, FDIM), jnp.float32)],
    )(xm, xd, ew2d, w1r, w1s, w2r, w2s, b1r, b2r)


def _stage_c(agg1, root1, agg2, root2, w3r, w3s, b3r):
    return pl.pallas_call(
        _stage_c_body,
        grid=(_GRID,),
        in_specs=[_node_spec, _node_spec, _node_spec, _node_spec,
                  _w_spec, _w_spec, _b_spec],
        out_specs=[_node_spec, _node_spec],
        out_shape=[_node_sds, _node_sds],
    )(agg1, root1, agg2, root2, w3r, w3s, b3r)


def _stage_e(agg3, root3, wl, bl):
    return pl.pallas_call(
        _stage_e_body,
        grid=(_GRID,),
        in_specs=[pl.BlockSpec((2, _ROWB, FDIM), lambda i: (0, i, 0)),
                  _node_spec, _w_spec, _b_spec],
        out_specs=_node_spec,
        out_shape=_node_sds,
    )(agg3, root3, wl, bl)


def _zero_acc(zbuf, acc, s):
    def zrow(i, carry):
        for k in range(FDIM // 16):
            zbuf[i, pl.ds(16 * k, 16)] = jnp.zeros((16,), jnp.float32)
        return carry
    lax.fori_loop(0, ZROWS, zrow, 0)
    for j in range(ROWS_PER_TILE // ZROWS):
        pltpu.sync_copy(zbuf, acc.at[pl.ds(s * ROWS_PER_TILE + j * ZROWS, ZROWS)])


def _conv_chunk(tab, srcr, dstr, wr, weighted, src_v, dst_v, w_v, rows_v, acc, sem, g):
    off = g * CHUNK
    pltpu.sync_copy(srcr.at[pl.ds(off, CHUNK)], src_v)
    pltpu.sync_copy(dstr.at[pl.ds(off, CHUNK)], dst_v)
    pltpu.async_copy(tab.at[src_v], rows_v, sem).wait()
    if weighted:
        pltpu.sync_copy(wr.at[pl.ds(off, CHUNK)], w_v)

        def scale(i, carry):
            wvec = plsc.load_gather(w_v, [jnp.full((16,), i, jnp.int32)])
            for k in range(FDIM // 16):
                rows_v[i, pl.ds(16 * k, 16)] = rows_v[i, pl.ds(16 * k, 16)] * wvec
            return carry
        lax.fori_loop(0, CHUNK, scale, 0)
    pltpu.sync_copy(rows_v, acc.at[dst_v], add=True)


def _run_conv(tab, srcr, dstr, wr, weighted, out, scratches, s, start, stride):
    src_v, dst_v, w_v, rows_v, zbuf, acc, sem = scratches
    _zero_acc(zbuf, acc, s)
    plsc.subcore_barrier()

    niter = (NUM_CHUNKS + stride - 1) // stride

    def body(i, carry):
        g = start + i * stride

        @pl.when(g < NUM_CHUNKS)
        def _():
            _conv_chunk(tab, srcr, dstr, wr, weighted,
                        src_v, dst_v, w_v, rows_v, acc, sem, g)
        return carry
    lax.fori_loop(0, niter, body, 0)
    plsc.subcore_barrier()
    pltpu.sync_copy(acc.at[pl.ds(s * ROWS_PER_TILE, ROWS_PER_TILE)],
                    out.at[pl.ds(s * ROWS_PER_TILE, ROWS_PER_TILE)])


_SC_SCRATCH = [
    pltpu.VMEM((CHUNK,), jnp.int32),
    pltpu.VMEM((CHUNK,), jnp.int32),
    pltpu.VMEM((CHUNK,), jnp.float32),
    pltpu.VMEM((CHUNK, FDIM), jnp.float32),
    pltpu.VMEM((ZROWS, FDIM), jnp.float32),
    pltpu.VMEM_SHARED((N_NODES, FDIM), jnp.float32),
    pltpu.SemaphoreType.DMA,
]

_sc_mesh = plsc.VectorSubcoreMesh(core_axis_name="c", subcore_axis_name="s")


@functools.partial(
    pl.kernel, mesh=_sc_mesh,
    out_type=[_node_sds, _node_sds],
    scratch_types=_SC_SCRATCH,
)
def _sc_stage_b(tab1, src1, dst1, tab2, src2, dst2, ew,
                out1, out2, *scratches):
    c = lax.axis_index("c")
    s = lax.axis_index("s")

    @pl.when(c == 0)
    def _():
        _run_conv(tab1, src1, dst1, None, False, out1, scratches,
                  s, s, NUM_SUBCORES)

    @pl.when(c == 1)
    def _():
        _run_conv(tab2, src2, dst2, ew, True, out2, scratches,
                  s, s, NUM_SUBCORES)


@functools.partial(
    pl.kernel, mesh=_sc_mesh,
    out_type=jax.ShapeDtypeStruct((2, N_NODES, FDIM), jnp.float32),
    scratch_types=_SC_SCRATCH,
)
def _sc_stage_d(tab, src, dst, ew, out, *scratches):
    c = lax.axis_index("c")
    s = lax.axis_index("s")
    _run_conv(tab, src, dst, ew, True, out.at[c], scratches,
              s, s * NUM_CORES + c, NUM_SUBCORES * NUM_CORES)


def kernel(x_measurement, x_demand, edge_index_mm, edge_index_md, edge_weight,
           W1r, b1r, W1s, W2r, b2r, W2s, W3r, b3r, W3s, Wl, bl):
    src_mm = edge_index_mm[0]
    dst_mm = edge_index_mm[1]
    src_md = edge_index_md[0]
    dst_md = edge_index_md[1]
    ew2d = edge_weight.reshape(NUM_EDGES // FDIM, FDIM)

    ym1, ym2, root1, root2, ew2d = _stage_a(
        x_measurement, x_demand, ew2d, W1r, W1s, W2r, W2s,
        b1r.reshape(1, FDIM), b2r.reshape(1, FDIM))
    ew = ew2d.reshape(-1)

    agg1, agg2 = _sc_stage_b(ym1, src_mm, dst_mm, ym2, src_md, dst_md, ew)

    ym3, root3 = _stage_c(agg1, root1, agg2, root2, W3r, W3s,
                          b3r.reshape(1, FDIM))

    agg3 = _sc_stage_d(ym3, src_md, dst_md, ew)

    return _stage_e(agg3, root3, Wl, bl.reshape(1, FDIM))


# speedup vs baseline: 4.3781x; 4.3781x over previous
import functools

import jax
import jax.numpy as jnp
from jax import lax
from jax.experimental import pallas as pl
from jax.experimental.pallas import tpu as pltpu
from jax.experimental.pallas import tpu_sc as plsc

N_NODES = 10000
FDIM = 128
NUM_EDGES = 320000
CHUNK = 128
NUM_CHUNKS = NUM_EDGES // CHUNK
NUM_CORES = 2
NUM_SUBCORES = 16
ROWS_PER_TILE = 624
ZROWS = 208

_DOT_DIMS = (((1,), (1,)), ((), ()))


def _dot(a, w):
    return lax.dot_general(a, w, _DOT_DIMS, preferred_element_type=jnp.float32)


def _stage_a_body(xm, xd, ewin, w1r, w1s, w2r, w2s, b1r, b2r,
                  ym1, ym2, root1, root2, ew):
    xmb = xm[...]
    xdb = xd[...]
    ym1[...] = _dot(xmb, w1r[...])
    ym2[...] = _dot(xmb, w2r[...])
    root1[...] = _dot(xmb, w1s[...]) + b1r[...]
    root2[...] = _dot(xdb, w2s[...]) + b2r[...]
    ew[...] = jax.nn.sigmoid(ewin[...])


def _stage_c_body(agg1, root1, agg2, root2, w3r, w3s, b3r, ym3, root3):
    movie = jnp.maximum(agg1[...] + root1[...], 0.0)
    ym3[...] = _dot(movie, w3r[...])
    user = jnp.maximum(agg2[...] + root2[...], 0.0)
    root3[...] = _dot(user, w3s[...]) + b3r[...]


def _stage_e_body(agg3, root3, wl, bl, out):
    a3 = agg3[...]
    user = jnp.maximum(a3[0] + a3[1] + root3[...], 0.0)
    out[...] = _dot(user, wl[...]) + bl[...]


_GRID = 10
_ROWB = N_NODES // _GRID

_node_spec = pl.BlockSpec((_ROWB, FDIM), lambda i: (i, 0))
_w_spec = pl.BlockSpec((FDIM, FDIM), lambda i: (0, 0))
_b_spec = pl.BlockSpec((1, FDIM), lambda i: (0, 0))
_ew_spec = pl.BlockSpec((256, FDIM), lambda i: (i, 0))
_node_sds = jax.ShapeDtypeStruct((N_NODES, FDIM), jnp.float32)


def _stage_a(xm, xd, ew2d, w1r, w1s, w2r, w2s, b1r, b2r):
    return pl.pallas_call(
        _stage_a_body,
        grid=(_GRID,),
        in_specs=[_node_spec, _node_spec, _ew_spec,
                  _w_spec, _w_spec, _w_spec, _w_spec, _b_spec, _b_spec],
        out_specs=[_node_spec, _node_spec, _node_spec, _node_spec, _ew_spec],
        out_shape=[_node_sds, _node_sds, _node_sds, _node_sds,
                   jax.ShapeDtypeStruct((NUM_EDGES // FDIM, FDIM), jnp.float32)],
    )(xm, xd, ew2d, w1r, w1s, w2r, w2s, b1r, b2r)


def _stage_c(agg1, root1, agg2, root2, w3r, w3s, b3r):
    return pl.pallas_call(
        _stage_c_body,
        grid=(_GRID,),
        in_specs=[_node_spec, _node_spec, _node_spec, _node_spec,
                  _w_spec, _w_spec, _b_spec],
        out_specs=[_node_spec, _node_spec],
        out_shape=[_node_sds, _node_sds],
    )(agg1, root1, agg2, root2, w3r, w3s, b3r)


def _stage_e(agg3, root3, wl, bl):
    return pl.pallas_call(
        _stage_e_body,
        grid=(_GRID,),
        in_specs=[pl.BlockSpec((2, _ROWB, FDIM), lambda i: (0, i, 0)),
                  _node_spec, _w_spec, _b_spec],
        out_specs=_node_spec,
        out_shape=_node_sds,
    )(agg3, root3, wl, bl)


def _zero_acc(zbuf, acc, s):
    def zrow(i, carry):
        for k in range(FDIM // 16):
            zbuf[i, pl.ds(16 * k, 16)] = jnp.zeros((16,), jnp.float32)
        return carry
    lax.fori_loop(0, ZROWS, zrow, 0)
    for j in range(ROWS_PER_TILE // ZROWS):
        pltpu.sync_copy(zbuf, acc.at[pl.ds(s * ROWS_PER_TILE + j * ZROWS, ZROWS)])

    @pl.when(s == NUM_SUBCORES - 1)
    def _():
        tail = N_NODES - NUM_SUBCORES * ROWS_PER_TILE
        pltpu.sync_copy(zbuf.at[pl.ds(0, tail)],
                        acc.at[pl.ds(NUM_SUBCORES * ROWS_PER_TILE, tail)])


def _conv_chunk(tab, srcr, dstr, wr, weighted,
                src_v, dst_v, w_v, rows_v, acc, sem, g):
    off = g * CHUNK
    pltpu.sync_copy(srcr.at[pl.ds(off, CHUNK)], src_v)
    pltpu.sync_copy(dstr.at[pl.ds(off, CHUNK)], dst_v)
    pltpu.async_copy(tab.at[src_v], rows_v, sem).wait()
    if weighted:
        pltpu.sync_copy(wr.at[pl.ds(off, CHUNK)], w_v)

        def scale(j, carry):
            wv = w_v[pl.ds(j * 16, 16)]
            for l in range(16):
                wvec = lax.full((16,), wv[l], jnp.float32)
                row = j * 16 + l
                for k in range(FDIM // 16):
                    rows_v[row, pl.ds(16 * k, 16)] = (
                        rows_v[row, pl.ds(16 * k, 16)] * wvec)
            return carry
        lax.fori_loop(0, CHUNK // 16, scale, 0)
    pltpu.sync_copy(rows_v, acc.at[dst_v], add=True)


def _run_conv(tab, srcr, dstr, wr, weighted, out, scratches, s, start, stride):
    src_v, dst_v, w_v, rows_v, zbuf, acc, sem = scratches
    _zero_acc(zbuf, acc, s)
    plsc.subcore_barrier()

    niter = (NUM_CHUNKS + stride - 1) // stride

    def body(i, carry):
        g = start + i * stride

        @pl.when(g < NUM_CHUNKS)
        def _():
            _conv_chunk(tab, srcr, dstr, wr, weighted,
                        src_v, dst_v, w_v, rows_v, acc, sem, g)
        return carry
    lax.fori_loop(0, niter, body, 0)
    plsc.subcore_barrier()
    pltpu.sync_copy(acc.at[pl.ds(s * ROWS_PER_TILE, ROWS_PER_TILE)],
                    out.at[pl.ds(s * ROWS_PER_TILE, ROWS_PER_TILE)])

    @pl.when(s == NUM_SUBCORES - 1)
    def _():
        tail = N_NODES - NUM_SUBCORES * ROWS_PER_TILE
        base = NUM_SUBCORES * ROWS_PER_TILE
        pltpu.sync_copy(acc.at[pl.ds(base, tail)], out.at[pl.ds(base, tail)])


_SC_SCRATCH = [
    pltpu.VMEM((CHUNK,), jnp.int32),
    pltpu.VMEM((CHUNK,), jnp.int32),
    pltpu.VMEM((CHUNK,), jnp.float32),
    pltpu.VMEM((CHUNK, FDIM), jnp.float32),
    pltpu.VMEM((ZROWS, FDIM), jnp.float32),
    pltpu.VMEM_SHARED((N_NODES, FDIM), jnp.float32),
    pltpu.SemaphoreType.DMA,
]

@functools.lru_cache(maxsize=None)
def _build_sc_kernels():
    mesh = plsc.VectorSubcoreMesh(core_axis_name="c", subcore_axis_name="s")

    @functools.partial(
        pl.kernel, mesh=mesh,
        out_type=[_node_sds, _node_sds],
        scratch_types=_SC_SCRATCH,
    )
    def sc_stage_b(tab1, src1, dst1, tab2, src2, dst2, ew,
                   out1, out2, *scratches):
        c = lax.axis_index("c")
        s = lax.axis_index("s")

        @pl.when(c == 0)
        def _():
            _run_conv(tab1, src1, dst1, None, False, out1, scratches,
                      s, s, NUM_SUBCORES)

        @pl.when(c == 1)
        def _():
            _run_conv(tab2, src2, dst2, ew, True, out2, scratches,
                      s, s, NUM_SUBCORES)

    @functools.partial(
        pl.kernel, mesh=mesh,
        out_type=jax.ShapeDtypeStruct((2, N_NODES, FDIM), jnp.float32),
        scratch_types=_SC_SCRATCH,
    )
    def sc_stage_d(tab, src, dst, ew, out, *scratches):
        c = lax.axis_index("c")
        s = lax.axis_index("s")
        _run_conv(tab, src, dst, ew, True, out.at[c], scratches,
                  s, s * NUM_CORES + c, NUM_SUBCORES * NUM_CORES)

    return sc_stage_b, sc_stage_d


def kernel(x_measurement, x_demand, edge_index_mm, edge_index_md, edge_weight,
           W1r, b1r, W1s, W2r, b2r, W2s, W3r, b3r, W3s, Wl, bl):
    src_mm = edge_index_mm[0]
    dst_mm = edge_index_mm[1]
    src_md = edge_index_md[0]
    dst_md = edge_index_md[1]
    ew2d = edge_weight.reshape(NUM_EDGES // FDIM, FDIM)

    ym1, ym2, root1, root2, ew2d = _stage_a(
        x_measurement, x_demand, ew2d, W1r, W1s, W2r, W2s,
        b1r.reshape(1, FDIM), b2r.reshape(1, FDIM))
    ew = ew2d.reshape(-1)

    sc_stage_b, sc_stage_d = _build_sc_kernels()
    agg1, agg2 = sc_stage_b(ym1, src_mm, dst_mm, ym2, src_md, dst_md, ew)

    ym3, root3 = _stage_c(agg1, root1, agg2, root2, W3r, W3s,
                          b3r.reshape(1, FDIM))

    agg3 = sc_stage_d(ym3, src_md, dst_md, ew)

    return _stage_e(agg3, root3, Wl, bl.reshape(1, FDIM))
